# baseline (speedup 1.0000x reference)
import functools

import jax
import jax.numpy as jnp
from jax import lax
from jax.experimental import pallas as pl
from jax.experimental.pallas import tpu as pltpu
from jax.experimental.pallas import tpu_sc as plsc

B = 16384
V = 100000
D = 16
F = 26
C = 13
ED = F * D

_NC = 2
_NS = 16
_NW = _NC * _NS
_RPW = ED // _NW
_CHB = 4096
_NCB = B // _CHB

_sc_mesh = plsc.VectorSubcoreMesh(core_axis_name="c", subcore_axis_name="s")


@functools.partial(
    pl.kernel,
    mesh=_sc_mesh,
    out_type=jax.ShapeDtypeStruct((ED, B), jnp.float32),
    scratch_types=[
        pltpu.VMEM((V,), jnp.float32),
        pltpu.VMEM((B,), jnp.int32),
        pltpu.VMEM((_CHB,), jnp.float32),
    ],
    compiler_params=pltpu.CompilerParams(use_tc_tiling_on_sc=True,
                                         needs_layout_passes=False),
)
def _sc_gather(xcatT_hbm, tabT_hbm, out_hbm, row_v, idx_v, out_v):
    wid = lax.axis_index("s") * _NC + lax.axis_index("c")

    def row_task(t, f_prev):
        r = wid * _RPW + t
        f = r // D
        pltpu.sync_copy(tabT_hbm.at[r], row_v)

        @pl.when(f != f_prev)
        def _():
            pltpu.sync_copy(xcatT_hbm.at[f], idx_v)

        def b_chunk(cb, carry2):
            b0 = cb * _CHB

            def gather16(i, carry3):
                iv = idx_v[pl.ds(b0 + i * 16, 16)]
                out_v[pl.ds(i * 16, 16)] = plsc.load_gather(row_v, [iv])
                return carry3

            lax.fori_loop(0, _CHB // 16, gather16, 0, unroll=4)
            pltpu.sync_copy(out_v, out_hbm.at[r, pl.ds(b0, _CHB)])
            return carry2

        lax.fori_loop(0, _NCB, b_chunk, 0)
        return f

    lax.fori_loop(0, _RPW, row_task, jnp.int32(-1))


_BM = 4096

_CN0 = (((0,), (0,)), ((), ()))


def _leaky(h):
    return jnp.where(h >= 0, h, 0.1 * h)


def _mlp_body(xeT_ref, xcT_ref, w1_ref, b1_ref, w2_ref, b2_ref,
              w3_ref, b3_ref, wmu_ref, bmu_ref, wlv_ref, blv_ref,
              xT_ref, muT_ref, lvT_ref):
    xeT = xeT_ref[...]
    xcT = xcT_ref[...]
    h = lax.dot_general(w1_ref[0:ED, :], xeT, _CN0,
                        preferred_element_type=jnp.float32)
    h = h + lax.dot_general(w1_ref[ED:, :], xcT, _CN0,
                            preferred_element_type=jnp.float32)
    h = _leaky(h + b1_ref[...])
    h = _leaky(lax.dot_general(w2_ref[...], h, _CN0,
                               preferred_element_type=jnp.float32) + b2_ref[...])
    h = _leaky(lax.dot_general(w3_ref[...], h, _CN0,
                               preferred_element_type=jnp.float32) + b3_ref[...])
    muT_ref[...] = lax.dot_general(wmu_ref[...], h, _CN0,
                                   preferred_element_type=jnp.float32) + bmu_ref[...]
    lvT_ref[...] = lax.dot_general(wlv_ref[...], h, _CN0,
                                   preferred_element_type=jnp.float32) + blv_ref[...]
    xT_ref[0:ED, :] = xeT
    xT_ref[ED:ED + C, :] = xcT


def _mlp(xeT, xcT, w1, b1c, w2, b2c, w3, b3c, wmu, bmuc, wlv, blvc):
    grid = (B // _BM,)
    col = lambda i: (0, i)
    rep = lambda i: (0, 0)
    return pl.pallas_call(
        _mlp_body,
        grid=grid,
        in_specs=[
            pl.BlockSpec((ED, _BM), col),
            pl.BlockSpec((C, _BM), col),
            pl.BlockSpec((ED + C, 256), rep),
            pl.BlockSpec((256, 1), rep),
            pl.BlockSpec((256, 128), rep),
            pl.BlockSpec((128, 1), rep),
            pl.BlockSpec((128, 64), rep),
            pl.BlockSpec((64, 1), rep),
            pl.BlockSpec((64, 32), rep),
            pl.BlockSpec((32, 1), rep),
            pl.BlockSpec((64, 32), rep),
            pl.BlockSpec((32, 1), rep),
        ],
        out_specs=[
            pl.BlockSpec((ED + C, _BM), col),
            pl.BlockSpec((32, _BM), col),
            pl.BlockSpec((32, _BM), col),
        ],
        out_shape=[
            jax.ShapeDtypeStruct((ED + C, B), jnp.float32),
            jax.ShapeDtypeStruct((32, B), jnp.float32),
            jax.ShapeDtypeStruct((32, B), jnp.float32),
        ],
    )(xeT, xcT, w1, b1c, w2, b2c, w3, b3c, wmu, bmuc, wlv, blvc)


def kernel(x_cont, x_cat, tables, W1, b1, W2, b2, W3, b3, Wmu, bmu, Wlv, blv):
    tabT = tables.transpose(0, 2, 1).reshape(ED, V)
    xcatT = x_cat.T
    xeT = _sc_gather(xcatT, tabT)
    xT, muT, lvT = _mlp(
        xeT, x_cont.T, W1,
        b1.reshape(-1, 1), W2, b2.reshape(-1, 1), W3, b3.reshape(-1, 1),
        Wmu, bmu.reshape(-1, 1), Wlv, blv.reshape(-1, 1),
    )
    return (muT.T, lvT.T, xT.T)


# speedup vs baseline: 1.7418x; 1.7418x over previous
import functools

import jax
import jax.numpy as jnp
from jax import lax
from jax.experimental import pallas as pl
from jax.experimental.pallas import tpu as pltpu
from jax.experimental.pallas import tpu_sc as plsc

B = 16384
V = 100000
D = 16
F = 26
C = 13
ED = F * D

_NC = 2
_NS = 16
_NW = _NC * _NS
_RPW = ED // _NW
_CHB = 4096
_NCB = B // _CHB

_sc_mesh = plsc.VectorSubcoreMesh(core_axis_name="c", subcore_axis_name="s")


@functools.partial(
    pl.kernel,
    mesh=_sc_mesh,
    out_type=jax.ShapeDtypeStruct((ED, B), jnp.float32),
    scratch_types=[
        pltpu.VMEM((V,), jnp.float32),
        pltpu.VMEM((B,), jnp.int32),
        pltpu.VMEM((_CHB,), jnp.float32),
    ],
    compiler_params=pltpu.CompilerParams(use_tc_tiling_on_sc=True,
                                         needs_layout_passes=False),
)
def _sc_gather(xcatT_hbm, tabT_hbm, out_hbm, row_v, idx_v, out_v):
    wid = lax.axis_index("s") * _NC + lax.axis_index("c")

    def row_task(t, f_prev):
        r = wid * _RPW + t
        f = r // D
        pltpu.sync_copy(tabT_hbm.at[r], row_v)

        @pl.when(f != f_prev)
        def _():
            pltpu.sync_copy(xcatT_hbm.at[f], idx_v)

        def b_chunk(cb, carry2):
            b0 = cb * _CHB

            @plsc.parallel_loop(0, _CHB, 16, unroll=4)
            def gather16(i):
                iv = idx_v[pl.ds(b0 + i, 16)]
                out_v[pl.ds(i, 16)] = plsc.load_gather(row_v, [iv])

            pltpu.sync_copy(out_v, out_hbm.at[r, pl.ds(b0, _CHB)])
            return carry2

        lax.fori_loop(0, _NCB, b_chunk, 0)
        return f

    lax.fori_loop(0, _RPW, row_task, jnp.int32(-1))


_BM = 4096

_CN0 = (((0,), (0,)), ((), ()))


def _leaky(h):
    return jnp.where(h >= 0, h, 0.1 * h)


def _mlp_body(xeT_ref, xcT_ref, w1_ref, b1_ref, w2_ref, b2_ref,
              w3_ref, b3_ref, wmu_ref, bmu_ref, wlv_ref, blv_ref,
              xT_ref, muT_ref, lvT_ref):
    xeT = xeT_ref[...]
    xcT = xcT_ref[...]
    h = lax.dot_general(w1_ref[0:ED, :], xeT, _CN0,
                        preferred_element_type=jnp.float32)
    h = h + lax.dot_general(w1_ref[ED:, :], xcT, _CN0,
                            preferred_element_type=jnp.float32)
    h = _leaky(h + b1_ref[...])
    h = _leaky(lax.dot_general(w2_ref[...], h, _CN0,
                               preferred_element_type=jnp.float32) + b2_ref[...])
    h = _leaky(lax.dot_general(w3_ref[...], h, _CN0,
                               preferred_element_type=jnp.float32) + b3_ref[...])
    muT_ref[...] = lax.dot_general(wmu_ref[...], h, _CN0,
                                   preferred_element_type=jnp.float32) + bmu_ref[...]
    lvT_ref[...] = lax.dot_general(wlv_ref[...], h, _CN0,
                                   preferred_element_type=jnp.float32) + blv_ref[...]
    xT_ref[0:ED, :] = xeT
    xT_ref[ED:ED + C, :] = xcT


def _mlp(xeT, xcT, w1, b1c, w2, b2c, w3, b3c, wmu, bmuc, wlv, blvc):
    grid = (B // _BM,)
    col = lambda i: (0, i)
    rep = lambda i: (0, 0)
    return pl.pallas_call(
        _mlp_body,
        grid=grid,
        in_specs=[
            pl.BlockSpec((ED, _BM), col),
            pl.BlockSpec((C, _BM), col),
            pl.BlockSpec((ED + C, 256), rep),
            pl.BlockSpec((256, 1), rep),
            pl.BlockSpec((256, 128), rep),
            pl.BlockSpec((128, 1), rep),
            pl.BlockSpec((128, 64), rep),
            pl.BlockSpec((64, 1), rep),
            pl.BlockSpec((64, 32), rep),
            pl.BlockSpec((32, 1), rep),
            pl.BlockSpec((64, 32), rep),
            pl.BlockSpec((32, 1), rep),
        ],
        out_specs=[
            pl.BlockSpec((ED + C, _BM), col),
            pl.BlockSpec((32, _BM), col),
            pl.BlockSpec((32, _BM), col),
        ],
        out_shape=[
            jax.ShapeDtypeStruct((ED + C, B), jnp.float32),
            jax.ShapeDtypeStruct((32, B), jnp.float32),
            jax.ShapeDtypeStruct((32, B), jnp.float32),
        ],
    )(xeT, xcT, w1, b1c, w2, b2c, w3, b3c, wmu, bmuc, wlv, blvc)


def kernel(x_cont, x_cat, tables, W1, b1, W2, b2, W3, b3, Wmu, bmu, Wlv, blv):
    tabT = tables.transpose(0, 2, 1).reshape(ED, V)
    xcatT = x_cat.T
    xeT = _sc_gather(xcatT, tabT)
    xT, muT, lvT = _mlp(
        xeT, x_cont.T, W1,
        b1.reshape(-1, 1), W2, b2.reshape(-1, 1), W3, b3.reshape(-1, 1),
        Wmu, bmu.reshape(-1, 1), Wlv, blv.reshape(-1, 1),
    )
    return (muT.T, lvT.T, xT.T)


# speedup vs baseline: 1.7589x; 1.0098x over previous
import functools

import jax
import jax.numpy as jnp
from jax import lax
from jax.experimental import pallas as pl
from jax.experimental.pallas import tpu as pltpu
from jax.experimental.pallas import tpu_sc as plsc

B = 16384
V = 100000
D = 16
F = 26
C = 13
ED = F * D

_NC = 2
_NS = 16
_NW = _NC * _NS
_RPW = ED // _NW
_CHB = 8192
_NCB = B // _CHB

_sc_mesh = plsc.VectorSubcoreMesh(core_axis_name="c", subcore_axis_name="s")


@functools.partial(
    pl.kernel,
    mesh=_sc_mesh,
    out_type=jax.ShapeDtypeStruct((ED, B), jnp.float32),
    scratch_types=[
        pltpu.VMEM((V,), jnp.float32),
        pltpu.VMEM((B,), jnp.int32),
        pltpu.VMEM((_CHB,), jnp.float32),
    ],
    compiler_params=pltpu.CompilerParams(use_tc_tiling_on_sc=True,
                                         needs_layout_passes=False),
)
def _sc_gather(xcatT_hbm, tabT_hbm, out_hbm, row_v, idx_v, out_v):
    wid = lax.axis_index("s") * _NC + lax.axis_index("c")

    def row_task(t, f_prev):
        r = wid * _RPW + t
        f = r // D
        pltpu.sync_copy(tabT_hbm.at[r], row_v)

        @pl.when(f != f_prev)
        def _():
            pltpu.sync_copy(xcatT_hbm.at[f], idx_v)

        def b_chunk(cb, carry2):
            b0 = cb * _CHB

            @plsc.parallel_loop(0, _CHB, 16, unroll=4)
            def gather16(i):
                iv = idx_v[pl.ds(b0 + i, 16)]
                out_v[pl.ds(i, 16)] = plsc.load_gather(row_v, [iv])

            pltpu.sync_copy(out_v, out_hbm.at[r, pl.ds(b0, _CHB)])
            return carry2

        lax.fori_loop(0, _NCB, b_chunk, 0)
        return f

    lax.fori_loop(0, _RPW, row_task, jnp.int32(-1))


_BM = 4096

_CN0 = (((0,), (0,)), ((), ()))


def _leaky(h):
    return jnp.where(h >= 0, h, 0.1 * h)


def _mlp_body(xeT_ref, xcT_ref, w1_ref, b1_ref, w2_ref, b2_ref,
              w3_ref, b3_ref, wmu_ref, bmu_ref, wlv_ref, blv_ref,
              xT_ref, muT_ref, lvT_ref):
    xeT = xeT_ref[...]
    xcT = xcT_ref[...]
    h = lax.dot_general(w1_ref[0:ED, :], xeT, _CN0,
                        preferred_element_type=jnp.float32)
    h = h + lax.dot_general(w1_ref[ED:, :], xcT, _CN0,
                            preferred_element_type=jnp.float32)
    h = _leaky(h + b1_ref[...])
    h = _leaky(lax.dot_general(w2_ref[...], h, _CN0,
                               preferred_element_type=jnp.float32) + b2_ref[...])
    h = _leaky(lax.dot_general(w3_ref[...], h, _CN0,
                               preferred_element_type=jnp.float32) + b3_ref[...])
    muT_ref[...] = lax.dot_general(wmu_ref[...], h, _CN0,
                                   preferred_element_type=jnp.float32) + bmu_ref[...]
    lvT_ref[...] = lax.dot_general(wlv_ref[...], h, _CN0,
                                   preferred_element_type=jnp.float32) + blv_ref[...]
    xT_ref[0:ED, :] = xeT
    xT_ref[ED:ED + C, :] = xcT


def _mlp(xeT, xcT, w1, b1c, w2, b2c, w3, b3c, wmu, bmuc, wlv, blvc):
    grid = (B // _BM,)
    col = lambda i: (0, i)
    rep = lambda i: (0, 0)
    return pl.pallas_call(
        _mlp_body,
        grid=grid,
        in_specs=[
            pl.BlockSpec((ED, _BM), col),
            pl.BlockSpec((C, _BM), col),
            pl.BlockSpec((ED + C, 256), rep),
            pl.BlockSpec((256, 1), rep),
            pl.BlockSpec((256, 128), rep),
            pl.BlockSpec((128, 1), rep),
            pl.BlockSpec((128, 64), rep),
            pl.BlockSpec((64, 1), rep),
            pl.BlockSpec((64, 32), rep),
            pl.BlockSpec((32, 1), rep),
            pl.BlockSpec((64, 32), rep),
            pl.BlockSpec((32, 1), rep),
        ],
        out_specs=[
            pl.BlockSpec((ED + C, _BM), col),
            pl.BlockSpec((32, _BM), col),
            pl.BlockSpec((32, _BM), col),
        ],
        out_shape=[
            jax.ShapeDtypeStruct((ED + C, B), jnp.float32),
            jax.ShapeDtypeStruct((32, B), jnp.float32),
            jax.ShapeDtypeStruct((32, B), jnp.float32),
        ],
    )(xeT, xcT, w1, b1c, w2, b2c, w3, b3c, wmu, bmuc, wlv, blvc)


def kernel(x_cont, x_cat, tables, W1, b1, W2, b2, W3, b3, Wmu, bmu, Wlv, blv):
    tabT = tables.transpose(0, 2, 1).reshape(ED, V)
    xcatT = x_cat.T
    xeT = _sc_gather(xcatT, tabT)
    xT, muT, lvT = _mlp(
        xeT, x_cont.T, W1,
        b1.reshape(-1, 1), W2, b2.reshape(-1, 1), W3, b3.reshape(-1, 1),
        Wmu, bmu.reshape(-1, 1), Wlv, blv.reshape(-1, 1),
    )
    return (muT.T, lvT.T, xT.T)


# speedup vs baseline: 1.7947x; 1.0204x over previous
import functools

import jax
import jax.numpy as jnp
from jax import lax
from jax.experimental import pallas as pl
from jax.experimental.pallas import tpu as pltpu
from jax.experimental.pallas import tpu_sc as plsc

B = 16384
V = 100000
D = 16
F = 26
C = 13
ED = F * D

_NC = 2
_NS = 16
_NW = _NC * _NS
_RPW = ED // _NW
_CHB = 8192
_NCB = B // _CHB

_sc_mesh = plsc.VectorSubcoreMesh(core_axis_name="c", subcore_axis_name="s")


@functools.partial(
    pl.kernel,
    mesh=_sc_mesh,
    out_type=jax.ShapeDtypeStruct((ED, B), jnp.float32),
    scratch_types=[
        pltpu.VMEM((V,), jnp.float32),
        pltpu.VMEM((B,), jnp.int32),
        pltpu.VMEM((_CHB,), jnp.float32),
    ],
    compiler_params=pltpu.CompilerParams(use_tc_tiling_on_sc=True,
                                         needs_layout_passes=False),
)
def _sc_gather(xcatT_hbm, tabT_hbm, out_hbm, row_v, idx_v, out_v):
    wid = lax.axis_index("s") * _NC + lax.axis_index("c")

    def row_task(t, f_prev):
        r = wid * _RPW + t
        f = r // D
        pltpu.sync_copy(tabT_hbm.at[r], row_v)

        @pl.when(f != f_prev)
        def _():
            pltpu.sync_copy(xcatT_hbm.at[f], idx_v)

        def b_chunk(cb, carry2):
            b0 = cb * _CHB

            @plsc.parallel_loop(0, _CHB, 16, unroll=8)
            def gather16(i):
                iv = idx_v[pl.ds(b0 + i, 16)]
                out_v[pl.ds(i, 16)] = plsc.load_gather(row_v, [iv])

            pltpu.sync_copy(out_v, out_hbm.at[r, pl.ds(b0, _CHB)])
            return carry2

        lax.fori_loop(0, _NCB, b_chunk, 0)
        return f

    lax.fori_loop(0, _RPW, row_task, jnp.int32(-1))


_BM = 4096

_CN0 = (((0,), (0,)), ((), ()))


def _leaky(h):
    return jnp.where(h >= 0, h, 0.1 * h)


def _mlp_body(xeT_ref, xcT_ref, w1_ref, b1_ref, w2_ref, b2_ref,
              w3_ref, b3_ref, wmu_ref, bmu_ref, wlv_ref, blv_ref,
              xT_ref, muT_ref, lvT_ref):
    xeT = xeT_ref[...]
    xcT = xcT_ref[...]
    h = lax.dot_general(w1_ref[0:ED, :], xeT, _CN0,
                        preferred_element_type=jnp.float32)
    h = h + lax.dot_general(w1_ref[ED:, :], xcT, _CN0,
                            preferred_element_type=jnp.float32)
    h = _leaky(h + b1_ref[...])
    h = _leaky(lax.dot_general(w2_ref[...], h, _CN0,
                               preferred_element_type=jnp.float32) + b2_ref[...])
    h = _leaky(lax.dot_general(w3_ref[...], h, _CN0,
                               preferred_element_type=jnp.float32) + b3_ref[...])
    muT_ref[...] = lax.dot_general(wmu_ref[...], h, _CN0,
                                   preferred_element_type=jnp.float32) + bmu_ref[...]
    lvT_ref[...] = lax.dot_general(wlv_ref[...], h, _CN0,
                                   preferred_element_type=jnp.float32) + blv_ref[...]
    xT_ref[0:ED, :] = xeT
    xT_ref[ED:ED + C, :] = xcT


def _mlp(xeT, xcT, w1, b1c, w2, b2c, w3, b3c, wmu, bmuc, wlv, blvc):
    grid = (B // _BM,)
    col = lambda i: (0, i)
    rep = lambda i: (0, 0)
    return pl.pallas_call(
        _mlp_body,
        grid=grid,
        in_specs=[
            pl.BlockSpec((ED, _BM), col),
            pl.BlockSpec((C, _BM), col),
            pl.BlockSpec((ED + C, 256), rep),
            pl.BlockSpec((256, 1), rep),
            pl.BlockSpec((256, 128), rep),
            pl.BlockSpec((128, 1), rep),
            pl.BlockSpec((128, 64), rep),
            pl.BlockSpec((64, 1), rep),
            pl.BlockSpec((64, 32), rep),
            pl.BlockSpec((32, 1), rep),
            pl.BlockSpec((64, 32), rep),
            pl.BlockSpec((32, 1), rep),
        ],
        out_specs=[
            pl.BlockSpec((ED + C, _BM), col),
            pl.BlockSpec((32, _BM), col),
            pl.BlockSpec((32, _BM), col),
        ],
        out_shape=[
            jax.ShapeDtypeStruct((ED + C, B), jnp.float32),
            jax.ShapeDtypeStruct((32, B), jnp.float32),
            jax.ShapeDtypeStruct((32, B), jnp.float32),
        ],
    )(xeT, xcT, w1, b1c, w2, b2c, w3, b3c, wmu, bmuc, wlv, blvc)


def kernel(x_cont, x_cat, tables, W1, b1, W2, b2, W3, b3, Wmu, bmu, Wlv, blv):
    tabT = tables.transpose(0, 2, 1).reshape(ED, V)
    xcatT = x_cat.T
    xeT = _sc_gather(xcatT, tabT)
    xT, muT, lvT = _mlp(
        xeT, x_cont.T, W1,
        b1.reshape(-1, 1), W2, b2.reshape(-1, 1), W3, b3.reshape(-1, 1),
        Wmu, bmu.reshape(-1, 1), Wlv, blv.reshape(-1, 1),
    )
    return (muT.T, lvT.T, xT.T)


# speedup vs baseline: 1.8019x; 1.0040x over previous
import functools

import jax
import jax.numpy as jnp
from jax import lax
from jax.experimental import pallas as pl
from jax.experimental.pallas import tpu as pltpu
from jax.experimental.pallas import tpu_sc as plsc

B = 16384
V = 100000
D = 16
F = 26
C = 13
ED = F * D

_NC = 2
_NS = 16
_NW = _NC * _NS
_RPW = ED // _NW
_CHB = 8192
_NCB = B // _CHB

_sc_mesh = plsc.VectorSubcoreMesh(core_axis_name="c", subcore_axis_name="s")


@functools.partial(
    pl.kernel,
    mesh=_sc_mesh,
    out_type=jax.ShapeDtypeStruct((ED, B), jnp.float32),
    scratch_types=[
        pltpu.VMEM((V,), jnp.float32),
        pltpu.VMEM((B,), jnp.int32),
        pltpu.VMEM((_CHB,), jnp.float32),
    ],
    compiler_params=pltpu.CompilerParams(use_tc_tiling_on_sc=True,
                                         needs_layout_passes=False),
)
def _sc_gather(xcatT_hbm, tabT_hbm, out_hbm, row_v, idx_v, out_v):
    wid = lax.axis_index("s") * _NC + lax.axis_index("c")

    def row_task(t, f_prev):
        r = wid * _RPW + t
        f = r // D
        pltpu.sync_copy(tabT_hbm.at[r], row_v)

        @pl.when(f != f_prev)
        def _():
            pltpu.sync_copy(xcatT_hbm.at[f], idx_v)

        def b_chunk(cb, carry2):
            b0 = cb * _CHB

            @plsc.parallel_loop(0, _CHB, 16, unroll=16)
            def gather16(i):
                iv = idx_v[pl.ds(b0 + i, 16)]
                out_v[pl.ds(i, 16)] = plsc.load_gather(row_v, [iv])

            pltpu.sync_copy(out_v, out_hbm.at[r, pl.ds(b0, _CHB)])
            return carry2

        lax.fori_loop(0, _NCB, b_chunk, 0)
        return f

    lax.fori_loop(0, _RPW, row_task, jnp.int32(-1))


_BM = 4096

_CN0 = (((0,), (0,)), ((), ()))


def _leaky(h):
    return jnp.where(h >= 0, h, 0.1 * h)


def _mlp_body(xeT_ref, xcT_ref, w1_ref, b1_ref, w2_ref, b2_ref,
              w3_ref, b3_ref, wmu_ref, bmu_ref, wlv_ref, blv_ref,
              xT_ref, muT_ref, lvT_ref):
    xeT = xeT_ref[...]
    xcT = xcT_ref[...]
    h = lax.dot_general(w1_ref[0:ED, :], xeT, _CN0,
                        preferred_element_type=jnp.float32)
    h = h + lax.dot_general(w1_ref[ED:, :], xcT, _CN0,
                            preferred_element_type=jnp.float32)
    h = _leaky(h + b1_ref[...])
    h = _leaky(lax.dot_general(w2_ref[...], h, _CN0,
                               preferred_element_type=jnp.float32) + b2_ref[...])
    h = _leaky(lax.dot_general(w3_ref[...], h, _CN0,
                               preferred_element_type=jnp.float32) + b3_ref[...])
    muT_ref[...] = lax.dot_general(wmu_ref[...], h, _CN0,
                                   preferred_element_type=jnp.float32) + bmu_ref[...]
    lvT_ref[...] = lax.dot_general(wlv_ref[...], h, _CN0,
                                   preferred_element_type=jnp.float32) + blv_ref[...]
    xT_ref[0:ED, :] = xeT
    xT_ref[ED:ED + C, :] = xcT


def _mlp(xeT, xcT, w1, b1c, w2, b2c, w3, b3c, wmu, bmuc, wlv, blvc):
    grid = (B // _BM,)
    col = lambda i: (0, i)
    rep = lambda i: (0, 0)
    return pl.pallas_call(
        _mlp_body,
        grid=grid,
        in_specs=[
            pl.BlockSpec((ED, _BM), col),
            pl.BlockSpec((C, _BM), col),
            pl.BlockSpec((ED + C, 256), rep),
            pl.BlockSpec((256, 1), rep),
            pl.BlockSpec((256, 128), rep),
            pl.BlockSpec((128, 1), rep),
            pl.BlockSpec((128, 64), rep),
            pl.BlockSpec((64, 1), rep),
            pl.BlockSpec((64, 32), rep),
            pl.BlockSpec((32, 1), rep),
            pl.BlockSpec((64, 32), rep),
            pl.BlockSpec((32, 1), rep),
        ],
        out_specs=[
            pl.BlockSpec((ED + C, _BM), col),
            pl.BlockSpec((32, _BM), col),
            pl.BlockSpec((32, _BM), col),
        ],
        out_shape=[
            jax.ShapeDtypeStruct((ED + C, B), jnp.float32),
            jax.ShapeDtypeStruct((32, B), jnp.float32),
            jax.ShapeDtypeStruct((32, B), jnp.float32),
        ],
    )(xeT, xcT, w1, b1c, w2, b2c, w3, b3c, wmu, bmuc, wlv, blvc)


def kernel(x_cont, x_cat, tables, W1, b1, W2, b2, W3, b3, Wmu, bmu, Wlv, blv):
    tabT = tables.transpose(0, 2, 1).reshape(ED, V)
    xcatT = x_cat.T
    xeT = _sc_gather(xcatT, tabT)
    xT, muT, lvT = _mlp(
        xeT, x_cont.T, W1,
        b1.reshape(-1, 1), W2, b2.reshape(-1, 1), W3, b3.reshape(-1, 1),
        Wmu, bmu.reshape(-1, 1), Wlv, blv.reshape(-1, 1),
    )
    return (muT.T, lvT.T, xT.T)
